# baseline (speedup 1.0000x reference)
import functools

import jax
import jax.numpy as jnp
from jax import lax
from jax.experimental import pallas as pl
from jax.experimental.pallas import tpu as pltpu
from jax.experimental.pallas import tpu_sc as plsc

EMB = 64
ROWS = 16384
COLS = 50

NC = 2
NS = 16
NW = NC * NS

NRB = 16
RB = ROWS // NRB
NQ = 4
Q = RB // NQ
ITEMS = COLS * NRB
IPW = ITEMS // NW
NJ = IPW * NQ


def _gather_body(idx_hbm, tbl_hbm, out_hbm,
                 idx_v, pidx_v, buf0, buf1, tb0, tb1,
                 gsem0, gsem1, wsem0, wsem1):
    bufs = (buf0, buf1)
    tbs = (tb0, tb1)
    gsems = (gsem0, gsem1)
    wsems = (wsem0, wsem1)
    wid = lax.axis_index("s") * NC + lax.axis_index("c")
    i0 = wid * IPW
    iota = lax.iota(jnp.int32, 16)

    def stage_item(item):
        c = item // NRB
        rb = item % NRB
        pltpu.sync_copy(idx_hbm.at[c, rb], idx_v)

        def prow(r, carry):
            for g in range(8):
                v = idx_v[r, pl.ds(16 * g, 16)]
                pidx_v[r, pl.ds(16 * g, 16)] = lax.shift_right_logical(v, 1)
            return carry

        lax.fori_loop(0, 8, prow, 0)

    def fire_gathers(q, b):
        for h in range(2):
            pltpu.async_copy(
                tbl_hbm.at[pidx_v.at[2 * q + h]],
                bufs[b].at[pl.ds(h * 128, 128)],
                gsems[b])

    def drain_gathers(b):
        for h in range(2):
            pltpu.make_async_copy(
                tbl_hbm.at[pidx_v.at[0]],
                bufs[b].at[pl.ds(h * 128, 128)],
                gsems[b]).wait()

    def transpose_quarter(q, b):
        buf = bufs[b]
        tbuf = tbs[b]
        rvecs = []
        cvecs = []
        for rg in range(16):
            v = idx_v[2 * q + rg // 8, pl.ds(16 * (rg % 8), 16)]
            off = lax.shift_left(lax.bitwise_and(v, 1), 6)
            rvecs.append(iota + 16 * rg)
            cvecs.append(off)

        def frow(f, carry):
            for rg in range(16):
                col = cvecs[rg] + f
                vals = plsc.load_gather(buf, [rvecs[rg], col])
                tbuf[f, pl.ds(16 * rg, 16)] = vals
            return carry

        lax.fori_loop(0, EMB, frow, 0)

    def fire_write(item, q, b):
        c = item // NRB
        rb = item % NRB
        pltpu.async_copy(
            tbs[b], out_hbm.at[c, :, pl.ds(rb * RB + q * Q, Q)], wsems[b])

    def drain_write(b):
        pltpu.make_async_copy(
            tbs[b], out_hbm.at[0, :, pl.ds(0, Q)], wsems[b]).wait()

    stage_item(i0)
    fire_gathers(0, 0)

    def item_body(t, carry):
        item = i0 + t
        for q in range(NQ):
            b = q % 2
            drain_gathers(b)

            if q >= 2:
                drain_write(b)
            else:
                @pl.when(t >= 1)
                def _():
                    drain_write(b)

            transpose_quarter(q, b)
            fire_write(item, q, b)

            if q == NQ - 1:
                @pl.when(t < IPW - 1)
                def _():
                    stage_item(item + 1)
                    fire_gathers(0, 1 - b)
            else:
                fire_gathers(q + 1, 1 - b)
        return carry

    lax.fori_loop(0, IPW, item_body, 0)
    drain_write(0)
    drain_write(1)


def kernel(src, table):
    idx = src.T.astype(jnp.int32).reshape(COLS, NRB, 8, 128)
    tbl = table.reshape(500000, 128)
    mesh = plsc.VectorSubcoreMesh(core_axis_name="c", subcore_axis_name="s")
    emb = functools.partial(
        pl.kernel,
        mesh=mesh,
        out_type=jax.ShapeDtypeStruct((COLS, EMB, ROWS), jnp.float32),
        scratch_types=[
            pltpu.VMEM((8, 128), jnp.int32),
            pltpu.VMEM((8, 128), jnp.int32),
            pltpu.VMEM((Q, 128), jnp.float32),
            pltpu.VMEM((Q, 128), jnp.float32),
            pltpu.VMEM((EMB, Q), jnp.float32),
            pltpu.VMEM((EMB, Q), jnp.float32),
            pltpu.SemaphoreType.DMA,
            pltpu.SemaphoreType.DMA,
            pltpu.SemaphoreType.DMA,
            pltpu.SemaphoreType.DMA,
        ],
        compiler_params=pltpu.CompilerParams(
            use_tc_tiling_on_sc=True, needs_layout_passes=False),
    )(_gather_body)
    out = emb(idx, tbl)
    return out.transpose(2, 0, 1)


# speedup vs baseline: 2.3604x; 2.3604x over previous
import functools

import jax
import jax.numpy as jnp
from jax import lax
from jax.experimental import pallas as pl
from jax.experimental.pallas import tpu as pltpu
from jax.experimental.pallas import tpu_sc as plsc

EMB = 64
ROWS = 16384
COLS = 50

NC = 2
NS = 16
NW = NC * NS

NRB = 16
RB = ROWS // NRB
NQ = 4
Q = RB // NQ
ITEMS = COLS * NRB
IPW = ITEMS // NW
NJ = IPW * NQ


def _gather_body(idx_hbm, tbl_hbm, out_hbm,
                 idx_v, pidx_v, buf0, buf1, tb0, tb1,
                 gsem0, gsem1, wsem0, wsem1):
    bufs = (buf0, buf1)
    tbs = (tb0, tb1)
    gsems = (gsem0, gsem1)
    wsems = (wsem0, wsem1)
    wid = lax.axis_index("s") * NC + lax.axis_index("c")
    i0 = wid * IPW
    iota = lax.iota(jnp.int32, 16)

    def stage_item(item):
        c = item // NRB
        rb = item % NRB
        pltpu.sync_copy(idx_hbm.at[c, rb], idx_v)

        def prow(r, carry):
            for g in range(8):
                v = idx_v[r, pl.ds(16 * g, 16)]
                pidx_v[r, pl.ds(16 * g, 16)] = lax.shift_right_logical(v, 1)
            return carry

        lax.fori_loop(0, 8, prow, 0)

    def fire_gathers(q, b):
        for h in range(2):
            pltpu.async_copy(
                tbl_hbm.at[pidx_v.at[2 * q + h]],
                bufs[b].at[pl.ds(h * 128, 128)],
                gsems[b])

    def drain_gathers(b):
        for h in range(2):
            pltpu.make_async_copy(
                tbl_hbm.at[pidx_v.at[0]],
                bufs[b].at[pl.ds(h * 128, 128)],
                gsems[b]).wait()

    def transpose_quarter(q, b):
        buf = bufs[b]
        tbuf = tbs[b]
        rvecs = []
        cvecs = []
        for rg in range(16):
            v = idx_v[2 * q + rg // 8, pl.ds(16 * (rg % 8), 16)]
            off = lax.shift_left(lax.bitwise_and(v, 1), 6)
            rvecs.append(iota + 16 * rg)
            cvecs.append(off)

        @functools.partial(plsc.parallel_loop, 0, EMB, unroll=2)
        def frow(f):
            vals = [plsc.load_gather(buf, [rvecs[rg], cvecs[rg] + f])
                    for rg in range(16)]
            for rg in range(16):
                tbuf[f, pl.ds(16 * rg, 16)] = vals[rg]

    def fire_write(item, q, b):
        c = item // NRB
        rb = item % NRB
        pltpu.async_copy(
            tbs[b], out_hbm.at[c, :, pl.ds(rb * RB + q * Q, Q)], wsems[b])

    def drain_write(b):
        pltpu.make_async_copy(
            tbs[b], out_hbm.at[0, :, pl.ds(0, Q)], wsems[b]).wait()

    stage_item(i0)
    fire_gathers(0, 0)

    def item_body(t, carry):
        item = i0 + t
        for q in range(NQ):
            b = q % 2
            drain_gathers(b)

            if q >= 2:
                drain_write(b)
            else:
                @pl.when(t >= 1)
                def _():
                    drain_write(b)

            transpose_quarter(q, b)
            fire_write(item, q, b)

            if q == NQ - 1:
                @pl.when(t < IPW - 1)
                def _():
                    stage_item(item + 1)
                    fire_gathers(0, 1 - b)
            else:
                fire_gathers(q + 1, 1 - b)
        return carry

    lax.fori_loop(0, IPW, item_body, 0)
    drain_write(0)
    drain_write(1)


def kernel(src, table):
    idx = src.T.astype(jnp.int32).reshape(COLS, NRB, 8, 128)
    tbl = table.reshape(500000, 128)
    mesh = plsc.VectorSubcoreMesh(core_axis_name="c", subcore_axis_name="s")
    emb = functools.partial(
        pl.kernel,
        mesh=mesh,
        out_type=jax.ShapeDtypeStruct((COLS, EMB, ROWS), jnp.float32),
        scratch_types=[
            pltpu.VMEM((8, 128), jnp.int32),
            pltpu.VMEM((8, 128), jnp.int32),
            pltpu.VMEM((Q, 128), jnp.float32),
            pltpu.VMEM((Q, 128), jnp.float32),
            pltpu.VMEM((EMB, Q), jnp.float32),
            pltpu.VMEM((EMB, Q), jnp.float32),
            pltpu.SemaphoreType.DMA,
            pltpu.SemaphoreType.DMA,
            pltpu.SemaphoreType.DMA,
            pltpu.SemaphoreType.DMA,
        ],
        compiler_params=pltpu.CompilerParams(
            use_tc_tiling_on_sc=True, needs_layout_passes=False),
    )(_gather_body)
    out = emb(idx, tbl)
    return out.transpose(2, 0, 1)
